# baseline (speedup 1.0000x reference)
import functools
import math

import jax
import jax.numpy as jnp
from jax import lax
from jax.experimental import pallas as pl
from jax.experimental.pallas import tpu as pltpu
from jax.experimental.pallas import tpu_sc as plsc

VOCAB = 100000
D_MODEL = 768
SCALE = math.sqrt(D_MODEL)
LANES = 16

_B = 1024 * 50
_NW = 32
_B_PER_W = _B // _NW
_CHUNK = 64
_N_CHUNKS = _B_PER_W // _CHUNK


def _emb_kernel(idx_hbm, table_hbm, out_hbm, idx_v, rows_v, sem):
    nc = 2
    wid = lax.axis_index("s") * nc + lax.axis_index("c")
    base = wid * _B_PER_W

    def chunk_body(g, _):
        row0 = pl.multiple_of(base + g * _CHUNK, _CHUNK)
        pltpu.sync_copy(idx_hbm.at[pl.ds(row0, _CHUNK)], idx_v)
        pltpu.async_copy(table_hbm.at[idx_v], rows_v, sem).wait()

        def row_body(r, _):
            for j in range(D_MODEL // LANES):
                sl = pl.ds(j * LANES, LANES)
                rows_v[r, sl] = rows_v[r, sl] * SCALE
            return 0

        lax.fori_loop(0, _CHUNK, row_body, 0)

        pltpu.sync_copy(rows_v, out_hbm.at[pl.ds(row0, _CHUNK)])
        return 0

    lax.fori_loop(0, _N_CHUNKS, chunk_body, 0)


@jax.jit
def _emb(x_flat, lut_weight):
    mesh = plsc.VectorSubcoreMesh(core_axis_name="c", subcore_axis_name="s")
    run = pl.kernel(
        _emb_kernel,
        out_type=jax.ShapeDtypeStruct((_B, D_MODEL), jnp.float32),
        mesh=mesh,
        scratch_types=[
            pltpu.VMEM((_CHUNK,), jnp.int32),
            pltpu.VMEM((_CHUNK, D_MODEL), jnp.float32),
            pltpu.SemaphoreType.DMA,
        ],
    )
    return run(x_flat, lut_weight)


def kernel(x, lut_weight):
    x_flat = x.reshape(-1).astype(jnp.int32)
    out = _emb(x_flat, lut_weight)
    return out.reshape(x.shape + (D_MODEL,))


# speedup vs baseline: 1.0914x; 1.0914x over previous
import math

import jax
import jax.numpy as jnp
from jax import lax
from jax.experimental import pallas as pl
from jax.experimental.pallas import tpu as pltpu
from jax.experimental.pallas import tpu_sc as plsc

VOCAB = 100000
D_MODEL = 768
SCALE = math.sqrt(D_MODEL)
LANES = 16

_B = 1024 * 50
_NW = 32
_B_PER_W = _B // _NW
_CHUNK = 80
_N_CHUNKS = _B_PER_W // _CHUNK


def _emb_kernel(idx_hbm, table_hbm, out_hbm, idx_all, rows_a, rows_b,
                gsem_a, gsem_b, wsem_a, wsem_b):
    nc = 2
    wid = lax.axis_index("s") * nc + lax.axis_index("c")
    base = wid * _B_PER_W

    def gather_start(g, rows_v, gsem):
        off = pl.multiple_of(g * _CHUNK, 8)
        pltpu.async_copy(table_hbm.at[idx_all.at[pl.ds(off, _CHUNK)]],
                         rows_v, gsem)

    def gather_wait(rows_v, gsem):
        pltpu.make_async_copy(table_hbm.at[idx_all.at[pl.ds(0, _CHUNK)]],
                              rows_v, gsem).wait()

    def write_start(g, rows_v, wsem):
        off = pl.multiple_of(base + g * _CHUNK, 8)
        pltpu.async_copy(rows_v, out_hbm.at[pl.ds(off, _CHUNK)], wsem)

    def write_wait(rows_v, wsem):
        pltpu.make_async_copy(rows_v, out_hbm.at[pl.ds(base, _CHUNK)],
                              wsem).wait()

    def scale_buf(rows_v):
        def row_body(r, _):
            for j in range(D_MODEL // LANES):
                sl = pl.ds(j * LANES, LANES)
                rows_v[r, sl] = rows_v[r, sl] * SCALE
            return 0

        lax.fori_loop(0, _CHUNK, row_body, 0)

    pltpu.sync_copy(idx_hbm.at[pl.ds(base, _B_PER_W)], idx_all)

    gather_start(0, rows_a, gsem_a)
    gather_start(1, rows_b, gsem_b)

    def pair_body(k, _):
        g0 = 2 * k
        gather_wait(rows_a, gsem_a)
        scale_buf(rows_a)
        write_start(g0, rows_a, wsem_a)

        gather_wait(rows_b, gsem_b)
        scale_buf(rows_b)
        write_start(g0 + 1, rows_b, wsem_b)

        write_wait(rows_a, wsem_a)

        @pl.when(g0 + 2 < _N_CHUNKS)
        def _():
            gather_start(g0 + 2, rows_a, gsem_a)

        write_wait(rows_b, wsem_b)

        @pl.when(g0 + 3 < _N_CHUNKS)
        def _():
            gather_start(g0 + 3, rows_b, gsem_b)

        return 0

    lax.fori_loop(0, _N_CHUNKS // 2, pair_body, 0)


@jax.jit
def _emb(x_flat, lut_weight):
    mesh = plsc.VectorSubcoreMesh(core_axis_name="c", subcore_axis_name="s")
    run = pl.kernel(
        _emb_kernel,
        out_type=jax.ShapeDtypeStruct((_B, D_MODEL), jnp.float32),
        mesh=mesh,
        scratch_types=[
            pltpu.VMEM((_B_PER_W,), jnp.int32),
            pltpu.VMEM((_CHUNK, D_MODEL), jnp.float32),
            pltpu.VMEM((_CHUNK, D_MODEL), jnp.float32),
            pltpu.SemaphoreType.DMA,
            pltpu.SemaphoreType.DMA,
            pltpu.SemaphoreType.DMA,
            pltpu.SemaphoreType.DMA,
        ],
    )
    return run(x_flat, lut_weight)


def kernel(x, lut_weight):
    x_flat = x.reshape(-1).astype(jnp.int32)
    out = _emb(x_flat, lut_weight)
    return out.reshape(x.shape + (D_MODEL,))


# speedup vs baseline: 1.0925x; 1.0010x over previous
import math

import jax
import jax.numpy as jnp
from jax import lax
from jax.experimental import pallas as pl
from jax.experimental.pallas import tpu as pltpu
from jax.experimental.pallas import tpu_sc as plsc

VOCAB = 100000
D_MODEL = 768
SCALE = math.sqrt(D_MODEL)
LANES = 16

_B = 1024 * 50
_NW = 32
_B_PER_W = _B // _NW
_CHUNK = 80
_N_CHUNKS = _B_PER_W // _CHUNK


def _emb_kernel(idx_hbm, table_hbm, out_hbm, idx_all, rows_a, rows_b,
                gsem_a, gsem_b, wsem_a, wsem_b):
    nc = 2
    wid = lax.axis_index("s") * nc + lax.axis_index("c")
    base = wid * _B_PER_W

    def gather_start(g, rows_v, gsem):
        off = pl.multiple_of(g * _CHUNK, 8)
        pltpu.async_copy(table_hbm.at[idx_all.at[pl.ds(off, _CHUNK)]],
                         rows_v, gsem)

    def gather_wait(rows_v, gsem):
        pltpu.make_async_copy(table_hbm.at[idx_all.at[pl.ds(0, _CHUNK)]],
                              rows_v, gsem).wait()

    def write_start(g, rows_v, wsem):
        off = pl.multiple_of(base + g * _CHUNK, 8)
        pltpu.async_copy(rows_v, out_hbm.at[pl.ds(off, _CHUNK)], wsem)

    def write_wait(rows_v, wsem):
        pltpu.make_async_copy(rows_v, out_hbm.at[pl.ds(base, _CHUNK)],
                              wsem).wait()

    def scale_buf(rows_v):
        def row_body(r, _):
            for j in range(D_MODEL // LANES):
                sl = pl.ds(j * LANES, LANES)
                rows_v[r, sl] = rows_v[r, sl] * SCALE
            return 0

        lax.fori_loop(0, _CHUNK, row_body, 0)

    pltpu.sync_copy(idx_hbm.at[pl.ds(base, _B_PER_W)], idx_all)

    gather_start(0, rows_a, gsem_a)
    gather_start(1, rows_b, gsem_b)

    def pair_body(k, _):
        g0 = 2 * k
        gather_wait(rows_a, gsem_a)
        scale_buf(rows_a)
        write_start(g0, rows_a, wsem_a)

        gather_wait(rows_b, gsem_b)
        scale_buf(rows_b)
        write_start(g0 + 1, rows_b, wsem_b)

        write_wait(rows_a, wsem_a)

        @pl.when(g0 + 2 < _N_CHUNKS)
        def _():
            gather_start(g0 + 2, rows_a, gsem_a)

        write_wait(rows_b, wsem_b)

        @pl.when(g0 + 3 < _N_CHUNKS)
        def _():
            gather_start(g0 + 3, rows_b, gsem_b)

        return 0

    lax.fori_loop(0, _N_CHUNKS // 2, pair_body, 0)


@jax.jit
def _emb(x_flat, lut_weight):
    mesh = plsc.VectorSubcoreMesh(core_axis_name="c", subcore_axis_name="s")
    run = pl.kernel(
        _emb_kernel,
        out_type=jax.ShapeDtypeStruct((_B, D_MODEL), jnp.float32),
        mesh=mesh,
        compiler_params=pltpu.CompilerParams(use_tc_tiling_on_sc=True),
        scratch_types=[
            pltpu.VMEM((_B_PER_W,), jnp.int32),
            pltpu.VMEM((_CHUNK, D_MODEL), jnp.float32),
            pltpu.VMEM((_CHUNK, D_MODEL), jnp.float32),
            pltpu.SemaphoreType.DMA,
            pltpu.SemaphoreType.DMA,
            pltpu.SemaphoreType.DMA,
            pltpu.SemaphoreType.DMA,
        ],
    )
    return run(x_flat, lut_weight)


def kernel(x, lut_weight):
    x_flat = x.reshape(-1).astype(jnp.int32)
    out = _emb(x_flat, lut_weight)
    return out.reshape(x.shape + (D_MODEL,))


# speedup vs baseline: 2.9349x; 2.6864x over previous
import math

import jax
import jax.numpy as jnp
from jax import lax
from jax.experimental import pallas as pl
from jax.experimental.pallas import tpu as pltpu
from jax.experimental.pallas import tpu_sc as plsc

VOCAB = 100000
D_MODEL = 768
SCALE = math.sqrt(D_MODEL)
LANES = 16

_B = 1024 * 50
_NW = 32
_B_PER_W = _B // _NW
_CHUNK = 80
_N_CHUNKS = _B_PER_W // _CHUNK


def _emb_kernel(idx_hbm, table_hbm, out_hbm, idx_all, rows_a, rows_b,
                gsem_a, gsem_b, wsem_a, wsem_b):
    nc = 2
    wid = lax.axis_index("s") * nc + lax.axis_index("c")
    base = wid * _B_PER_W

    def gather_start(g, rows_v, gsem):
        off = pl.multiple_of(g * _CHUNK, 8)
        pltpu.async_copy(table_hbm.at[idx_all.at[pl.ds(off, _CHUNK)]],
                         rows_v, gsem)

    def gather_wait(rows_v, gsem):
        pltpu.make_async_copy(table_hbm.at[idx_all.at[pl.ds(0, _CHUNK)]],
                              rows_v, gsem).wait()

    def write_start(g, rows_v, wsem):
        off = pl.multiple_of(base + g * _CHUNK, 8)
        pltpu.async_copy(rows_v, out_hbm.at[pl.ds(off, _CHUNK)], wsem)

    def write_wait(rows_v, wsem):
        pltpu.make_async_copy(rows_v, out_hbm.at[pl.ds(base, _CHUNK)],
                              wsem).wait()

    def scale_buf(rows_v):
        def row_body(r, _):
            for j in range(D_MODEL // LANES):
                sl = pl.ds(j * LANES, LANES)
                rows_v[r, sl] = rows_v[r, sl] * SCALE
            return 0

        lax.fori_loop(0, _CHUNK, row_body, 0)

    pltpu.sync_copy(idx_hbm.at[pl.ds(base, _B_PER_W)], idx_all)

    gather_start(0, rows_a, gsem_a)
    gather_start(1, rows_b, gsem_b)

    def pair_body(k, _):
        g0 = 2 * k
        gather_wait(rows_a, gsem_a)
        scale_buf(rows_a)
        write_start(g0, rows_a, wsem_a)

        gather_wait(rows_b, gsem_b)
        scale_buf(rows_b)
        write_start(g0 + 1, rows_b, wsem_b)

        write_wait(rows_a, wsem_a)

        @pl.when(g0 + 2 < _N_CHUNKS)
        def _():
            gather_start(g0 + 2, rows_a, gsem_a)

        write_wait(rows_b, wsem_b)

        @pl.when(g0 + 3 < _N_CHUNKS)
        def _():
            gather_start(g0 + 3, rows_b, gsem_b)

        return 0

    lax.fori_loop(0, _N_CHUNKS // 2, pair_body, 0)


@jax.jit
def _emb(x_flat, lut_weight):
    mesh = plsc.VectorSubcoreMesh(core_axis_name="c", subcore_axis_name="s")
    run = pl.kernel(
        _emb_kernel,
        out_type=jax.ShapeDtypeStruct((_B, D_MODEL), jnp.float32),
        mesh=mesh,
        compiler_params=pltpu.CompilerParams(use_tc_tiling_on_sc=True),
        scratch_types=[
            pltpu.VMEM((_B_PER_W,), jnp.int32),
            pltpu.VMEM((_CHUNK, D_MODEL), jnp.float32),
            pltpu.VMEM((_CHUNK, D_MODEL), jnp.float32),
            pltpu.SemaphoreType.DMA,
            pltpu.SemaphoreType.DMA,
            pltpu.SemaphoreType.DMA,
            pltpu.SemaphoreType.DMA,
        ],
    )
    return run(x_flat, lut_weight)


def kernel(x, lut_weight):
    a, b = x.shape
    xt_flat = x.T.reshape(-1).astype(jnp.int32)
    out = _emb(xt_flat, lut_weight)
    return out.reshape(b, a, D_MODEL).transpose(1, 0, 2)


# speedup vs baseline: 3.4039x; 1.1598x over previous
import math

import jax
import jax.numpy as jnp
from jax import lax
from jax.experimental import pallas as pl
from jax.experimental.pallas import tpu as pltpu
from jax.experimental.pallas import tpu_sc as plsc

VOCAB = 100000
D_MODEL = 768
SCALE = math.sqrt(D_MODEL)
LANES = 16

_B = 1024 * 50
_NW = 32
_B_PER_W = _B // _NW
_CHUNK = 40
_N_CHUNKS = _B_PER_W // _CHUNK


def _emb_kernel(idx_hbm, table_hbm, out_hbm, idx_all, gbuf_a, gbuf_b,
                wbuf_a, wbuf_b, gsem_a, gsem_b, wsem_a, wsem_b):
    nc = 2
    wid = lax.axis_index("s") * nc + lax.axis_index("c")
    base = wid * _B_PER_W

    def gather_start(g, rows_v, gsem):
        off = pl.multiple_of(g * _CHUNK, 8)
        pltpu.async_copy(table_hbm.at[idx_all.at[pl.ds(off, _CHUNK)]],
                         rows_v, gsem)

    def gather_wait(rows_v, gsem):
        pltpu.make_async_copy(table_hbm.at[idx_all.at[pl.ds(0, _CHUNK)]],
                              rows_v, gsem).wait()

    def write_start(g, rows_v, wsem):
        off = pl.multiple_of(base + g * _CHUNK, 8)
        pltpu.async_copy(rows_v, out_hbm.at[pl.ds(off, _CHUNK)], wsem)

    def write_wait(rows_v, wsem):
        pltpu.make_async_copy(rows_v, out_hbm.at[pl.ds(base, _CHUNK)],
                              wsem).wait()

    def scale_into(src_v, dst_v):
        @plsc.parallel_loop(0, _CHUNK, step=1, unroll=2)
        def _(r):
            for j in range(D_MODEL // LANES):
                sl = pl.ds(j * LANES, LANES)
                dst_v[r, sl] = src_v[r, sl] * SCALE

    pltpu.sync_copy(idx_hbm.at[pl.ds(base, _B_PER_W)], idx_all)

    gather_start(0, gbuf_a, gsem_a)
    gather_start(1, gbuf_b, gsem_b)

    def pair_body(k, _):
        g0 = 2 * k

        def do_slot(g, gbuf, gsem, wbuf, wsem):
            gather_wait(gbuf, gsem)

            @pl.when(k > 0)
            def _():
                write_wait(wbuf, wsem)

            scale_into(gbuf, wbuf)
            write_start(g, wbuf, wsem)

            @pl.when(g + 2 < _N_CHUNKS)
            def _():
                gather_start(g + 2, gbuf, gsem)

        do_slot(g0, gbuf_a, gsem_a, wbuf_a, wsem_a)
        do_slot(g0 + 1, gbuf_b, gsem_b, wbuf_b, wsem_b)
        return 0

    lax.fori_loop(0, _N_CHUNKS // 2, pair_body, 0)

    write_wait(wbuf_a, wsem_a)
    write_wait(wbuf_b, wsem_b)


@jax.jit
def _emb(x_flat, lut_weight):
    mesh = plsc.VectorSubcoreMesh(core_axis_name="c", subcore_axis_name="s")
    run = pl.kernel(
        _emb_kernel,
        out_type=jax.ShapeDtypeStruct((_B, D_MODEL), jnp.float32),
        mesh=mesh,
        compiler_params=pltpu.CompilerParams(use_tc_tiling_on_sc=True),
        scratch_types=[
            pltpu.VMEM((_B_PER_W,), jnp.int32),
            pltpu.VMEM((_CHUNK, D_MODEL), jnp.float32),
            pltpu.VMEM((_CHUNK, D_MODEL), jnp.float32),
            pltpu.VMEM((_CHUNK, D_MODEL), jnp.float32),
            pltpu.VMEM((_CHUNK, D_MODEL), jnp.float32),
            pltpu.SemaphoreType.DMA,
            pltpu.SemaphoreType.DMA,
            pltpu.SemaphoreType.DMA,
            pltpu.SemaphoreType.DMA,
        ],
    )
    return run(x_flat, lut_weight)


def kernel(x, lut_weight):
    a, b = x.shape
    xt_flat = x.T.reshape(-1).astype(jnp.int32)
    out = _emb(xt_flat, lut_weight)
    return out.reshape(b, a, D_MODEL).transpose(1, 0, 2)
